# baseline (speedup 1.0000x reference)
import functools

import jax
import jax.numpy as jnp
from jax import lax
from jax.experimental import pallas as pl
from jax.experimental.pallas import tpu as pltpu
from jax.experimental.pallas import tpu_sc as plsc

N = 10000
D = 128
E = 320000
NC = 2
NS = 16
L = 16
CHUNK = 128
KA = 80
KB = 80
KTOT = NS * (KA + KB)
G = 8
EPAD = KTOT * CHUNK
NPAD = 10112
RPT = NPAD // NS

_mesh = plsc.VectorSubcoreMesh(core_axis_name="c", subcore_axis_name="s")


def _chunk_base(cid, sid):
    return jnp.where(cid == 0, sid * KA, NS * KA + sid * KB)


@functools.partial(
    pl.kernel,
    out_type=jax.ShapeDtypeStruct((NC, NPAD, D), jnp.float32),
    mesh=_mesh,
    scratch_types=[
        pltpu.VMEM((G, CHUNK), jnp.int32),
        pltpu.VMEM((CHUNK, D), jnp.float32),
        pltpu.VMEM_SHARED((NPAD, D), jnp.float32),
    ],
)
def _deg_kernel(col_hbm, out_hbm, coli_v, ones_v, accum_sh):
    cid = lax.axis_index("c")
    sid = lax.axis_index("s")
    kbase = pl.multiple_of(_chunk_base(cid, sid), G)
    kc = jnp.where(cid == 0, KA, KB)

    @pl.loop(0, CHUNK)
    def _(i):
        @pl.loop(0, D, step=L)
        def _(j):
            ones_v[i, pl.ds(j, L)] = jnp.zeros((L,), jnp.float32)

    @pl.loop(0, 4)
    def _(i):
        pltpu.sync_copy(ones_v,
                        accum_sh.at[pl.ds(sid * RPT + i * CHUNK, CHUNK)])
    pltpu.sync_copy(ones_v.at[pl.ds(0, RPT - 4 * CHUNK)],
                    accum_sh.at[pl.ds(sid * RPT + 4 * CHUNK, RPT - 4 * CHUNK)])
    plsc.subcore_barrier()

    @pl.loop(0, CHUNK)
    def _(i):
        @pl.loop(0, D, step=L)
        def _(j):
            ones_v[i, pl.ds(j, L)] = jnp.full((L,), 1.0, jnp.float32)

    @pl.loop(0, kc, step=G)
    def _(j):
        jo = pl.multiple_of(j, G)
        pltpu.sync_copy(col_hbm.at[pl.ds(kbase + jo, G)], coli_v)
        for u in range(G):
            pltpu.sync_copy(ones_v, accum_sh.at[coli_v.at[u]], add=True)

    plsc.subcore_barrier()
    pltpu.sync_copy(accum_sh.at[pl.ds(sid * RPT, RPT)],
                    out_hbm.at[cid, pl.ds(sid * RPT, RPT)])


@functools.partial(
    pl.kernel,
    out_type=jax.ShapeDtypeStruct((NC, NPAD, D), jnp.float32),
    mesh=_mesh,
    scratch_types=[
        pltpu.VMEM((G, CHUNK), jnp.int32),
        pltpu.VMEM((G, CHUNK), jnp.int32),
        pltpu.VMEM((2, CHUNK, D), jnp.float32),
        pltpu.VMEM_SHARED((NPAD, D), jnp.float32),
        pltpu.SemaphoreType.DMA((2,)),
        pltpu.SemaphoreType.DMA((2,)),
    ],
)
def _agg_kernel(g_hbm, row_hbm, col_hbm, out_hbm,
                rowi_v, coli_v, msg_v, accum_sh, gsem, ssem):
    cid = lax.axis_index("c")
    sid = lax.axis_index("s")
    kbase = pl.multiple_of(_chunk_base(cid, sid), G)
    kc = jnp.where(cid == 0, KA, KB)

    @pl.loop(0, CHUNK)
    def _(i):
        @pl.loop(0, D, step=L)
        def _(j):
            msg_v[0, i, pl.ds(j, L)] = jnp.zeros((L,), jnp.float32)

    @pl.loop(0, 4)
    def _(i):
        pltpu.sync_copy(msg_v.at[0],
                        accum_sh.at[pl.ds(sid * RPT + i * CHUNK, CHUNK)])
    pltpu.sync_copy(msg_v.at[0, pl.ds(0, RPT - 4 * CHUNK)],
                    accum_sh.at[pl.ds(sid * RPT + 4 * CHUNK, RPT - 4 * CHUNK)])
    plsc.subcore_barrier()

    @pl.loop(0, kc, step=G)
    def _(j):
        jo = pl.multiple_of(j, G)
        pltpu.sync_copy(row_hbm.at[pl.ds(kbase + jo, G)], rowi_v)
        pltpu.sync_copy(col_hbm.at[pl.ds(kbase + jo, G)], coli_v)

        def gather(u):
            return pltpu.async_copy(g_hbm.at[rowi_v.at[u]],
                                    msg_v.at[u % 2], gsem.at[u % 2])

        def scatter(u):
            return pltpu.async_copy(msg_v.at[u % 2],
                                    accum_sh.at[coli_v.at[u]],
                                    ssem.at[u % 2], add=True)

        cp = {}
        cp["g0"] = gather(0)
        cp["g1"] = gather(1)
        cp["g0"].wait(); cp["s0"] = scatter(0)
        cp["g1"].wait(); cp["s1"] = scatter(1)
        for u in range(2, G, 2):
            cp[f"s{u-2}"].wait(); cp[f"g{u}"] = gather(u)
            cp[f"s{u-1}"].wait(); cp[f"g{u+1}"] = gather(u + 1)
            cp[f"g{u}"].wait(); cp[f"s{u}"] = scatter(u)
            cp[f"g{u+1}"].wait(); cp[f"s{u+1}"] = scatter(u + 1)
        cp[f"s{G-2}"].wait()
        cp[f"s{G-1}"].wait()

    plsc.subcore_barrier()
    pltpu.sync_copy(accum_sh.at[pl.ds(sid * RPT, RPT)],
                    out_hbm.at[cid, pl.ds(sid * RPT, RPT)])


BM = 2000


def _tc_a_body(dp_ref, x_ref, w_ref, g_ref, dis_ref):
    deg = dp_ref[0, :, 0] + dp_ref[1, :, 0] + 1.0
    dis = lax.rsqrt(deg)
    h = lax.dot_general(x_ref[...], w_ref[...], (((1,), (1,)), ((), ())),
                        preferred_element_type=jnp.float32)
    g_ref[...] = h * dis[:, None]
    dis_ref[...] = dis[:, None]


def _tc_a(degparts, x, w1):
    return pl.pallas_call(
        _tc_a_body,
        grid=(N // BM,),
        in_specs=[
            pl.BlockSpec((NC, BM, D), lambda i: (0, i, 0)),
            pl.BlockSpec((BM, D), lambda i: (i, 0)),
            pl.BlockSpec((D, D), lambda i: (0, 0)),
        ],
        out_specs=[
            pl.BlockSpec((BM, D), lambda i: (i, 0)),
            pl.BlockSpec((BM, 1), lambda i: (i, 0)),
        ],
        out_shape=[
            jax.ShapeDtypeStruct((N, D), jnp.float32),
            jax.ShapeDtypeStruct((N, 1), jnp.float32),
        ],
    )(degparts, x, w1)


def _tc_b_body(p_ref, g_ref, dis_ref, b_ref, w_ref, g2_ref):
    s = (p_ref[0] + p_ref[1] + g_ref[...]) * dis_ref[...] + b_ref[...]
    s = jnp.maximum(s, 0.0)
    h2 = lax.dot_general(s, w_ref[...], (((1,), (1,)), ((), ())),
                         preferred_element_type=jnp.float32)
    g2_ref[...] = h2 * dis_ref[...]


def _tc_b(parts, g, dis, b1, w2):
    return pl.pallas_call(
        _tc_b_body,
        grid=(N // BM,),
        in_specs=[
            pl.BlockSpec((NC, BM, D), lambda i: (0, i, 0)),
            pl.BlockSpec((BM, D), lambda i: (i, 0)),
            pl.BlockSpec((BM, 1), lambda i: (i, 0)),
            pl.BlockSpec((1, D), lambda i: (0, 0)),
            pl.BlockSpec((D, D), lambda i: (0, 0)),
        ],
        out_specs=pl.BlockSpec((BM, D), lambda i: (i, 0)),
        out_shape=jax.ShapeDtypeStruct((N, D), jnp.float32),
    )(parts, g, dis, b1, w2)


def _tc_c_body(p_ref, g_ref, dis_ref, b_ref, out_ref):
    out_ref[...] = (p_ref[0] + p_ref[1] + g_ref[...]) * dis_ref[...] + b_ref[...]


def _tc_c(parts, g, dis, b2):
    return pl.pallas_call(
        _tc_c_body,
        grid=(N // BM,),
        in_specs=[
            pl.BlockSpec((NC, BM, D), lambda i: (0, i, 0)),
            pl.BlockSpec((BM, D), lambda i: (i, 0)),
            pl.BlockSpec((BM, 1), lambda i: (i, 0)),
            pl.BlockSpec((1, D), lambda i: (0, 0)),
        ],
        out_specs=pl.BlockSpec((BM, D), lambda i: (i, 0)),
        out_shape=jax.ShapeDtypeStruct((N, D), jnp.float32),
    )(parts, g, dis, b2)


def kernel(x, edge_index, W1, b1, W2, b2):
    row = edge_index[0].astype(jnp.int32)
    col = edge_index[1].astype(jnp.int32)
    row_t = jnp.concatenate(
        [row, jnp.zeros((EPAD - E,), jnp.int32)]).reshape(KTOT, CHUNK)
    col_t = jnp.concatenate(
        [col, jnp.full((EPAD - E,), N, jnp.int32)]).reshape(KTOT, CHUNK)

    degparts = _deg_kernel(col_t)
    g1, dis = _tc_a(degparts, x, W1)
    parts1 = _agg_kernel(g1, row_t, col_t)
    g2 = _tc_b(parts1, g1, dis, b1.reshape(1, D), W2)
    parts2 = _agg_kernel(g2, row_t, col_t)
    return _tc_c(parts2, g2, dis, b2.reshape(1, D))


# speedup vs baseline: 1.1336x; 1.1336x over previous
import functools

import jax
import jax.numpy as jnp
from jax import lax
from jax.experimental import pallas as pl
from jax.experimental.pallas import tpu as pltpu
from jax.experimental.pallas import tpu_sc as plsc

N = 10000
D = 128
E = 320000
NC = 2
NS = 16
L = 16
CHUNK = 128
NW = NC * NS
K = 80
G = 8
EPAD = NW * K * CHUNK
NPAD = 10112
RPT = NPAD // NS

_mesh = plsc.VectorSubcoreMesh(core_axis_name="c", subcore_axis_name="s")


@functools.partial(
    pl.kernel,
    out_type=jax.ShapeDtypeStruct((NC, NPAD, D), jnp.float32),
    mesh=_mesh,
    scratch_types=[
        pltpu.VMEM((G, CHUNK), jnp.int32),
        pltpu.VMEM((CHUNK, D), jnp.float32),
        pltpu.VMEM_SHARED((NPAD, D), jnp.float32),
    ],
)
def _deg_kernel(col_hbm, out_hbm, coli_v, ones_v, accum_sh):
    cid = lax.axis_index("c")
    sid = lax.axis_index("s")
    wid = sid * NC + cid

    @pl.loop(0, CHUNK)
    def _(i):
        @pl.loop(0, D, step=L)
        def _(j):
            ones_v[i, pl.ds(j, L)] = jnp.zeros((L,), jnp.float32)

    @pl.loop(0, 4)
    def _(i):
        pltpu.sync_copy(ones_v,
                        accum_sh.at[pl.ds(sid * RPT + i * CHUNK, CHUNK)])
    pltpu.sync_copy(ones_v.at[pl.ds(0, RPT - 4 * CHUNK)],
                    accum_sh.at[pl.ds(sid * RPT + 4 * CHUNK, RPT - 4 * CHUNK)])
    plsc.subcore_barrier()

    @pl.loop(0, CHUNK)
    def _(i):
        @pl.loop(0, D, step=L)
        def _(j):
            ones_v[i, pl.ds(j, L)] = jnp.full((L,), 1.0, jnp.float32)

    @pl.loop(0, K, step=G)
    def _(j):
        jo = pl.multiple_of(j, G)
        pltpu.sync_copy(col_hbm.at[wid, pl.ds(jo, G)], coli_v)
        for u in range(G):
            pltpu.sync_copy(ones_v, accum_sh.at[coli_v.at[u]], add=True)

    plsc.subcore_barrier()
    pltpu.sync_copy(accum_sh.at[pl.ds(sid * RPT, RPT)],
                    out_hbm.at[cid, pl.ds(sid * RPT, RPT)])


@functools.partial(
    pl.kernel,
    out_type=jax.ShapeDtypeStruct((NC, NPAD, D), jnp.float32),
    mesh=_mesh,
    scratch_types=[
        pltpu.VMEM((G, CHUNK), jnp.int32),
        pltpu.VMEM((G, CHUNK), jnp.int32),
        pltpu.VMEM((2, CHUNK, D), jnp.float32),
        pltpu.VMEM_SHARED((NPAD, D), jnp.float32),
        pltpu.SemaphoreType.DMA((2,)),
        pltpu.SemaphoreType.DMA((2,)),
    ],
)
def _agg_kernel(g_hbm, row_hbm, col_hbm, out_hbm,
                rowi_v, coli_v, msg_v, accum_sh, gsem, ssem):
    cid = lax.axis_index("c")
    sid = lax.axis_index("s")
    wid = sid * NC + cid

    @pl.loop(0, CHUNK)
    def _(i):
        @pl.loop(0, D, step=L)
        def _(j):
            msg_v[0, i, pl.ds(j, L)] = jnp.zeros((L,), jnp.float32)

    @pl.loop(0, 4)
    def _(i):
        pltpu.sync_copy(msg_v.at[0],
                        accum_sh.at[pl.ds(sid * RPT + i * CHUNK, CHUNK)])
    pltpu.sync_copy(msg_v.at[0, pl.ds(0, RPT - 4 * CHUNK)],
                    accum_sh.at[pl.ds(sid * RPT + 4 * CHUNK, RPT - 4 * CHUNK)])
    plsc.subcore_barrier()

    @pl.loop(0, K, step=G)
    def _(j):
        jo = pl.multiple_of(j, G)
        pltpu.sync_copy(row_hbm.at[wid, pl.ds(jo, G)], rowi_v)
        pltpu.sync_copy(col_hbm.at[wid, pl.ds(jo, G)], coli_v)

        def gather(u):
            return pltpu.async_copy(g_hbm.at[rowi_v.at[u]],
                                    msg_v.at[u % 2], gsem.at[u % 2])

        def scatter(u):
            return pltpu.async_copy(msg_v.at[u % 2],
                                    accum_sh.at[coli_v.at[u]],
                                    ssem.at[u % 2], add=True)

        cp = {}
        cp["g0"] = gather(0)
        cp["g1"] = gather(1)
        cp["g0"].wait(); cp["s0"] = scatter(0)
        cp["g1"].wait(); cp["s1"] = scatter(1)
        for u in range(2, G, 2):
            cp[f"s{u-2}"].wait(); cp[f"g{u}"] = gather(u)
            cp[f"s{u-1}"].wait(); cp[f"g{u+1}"] = gather(u + 1)
            cp[f"g{u}"].wait(); cp[f"s{u}"] = scatter(u)
            cp[f"g{u+1}"].wait(); cp[f"s{u+1}"] = scatter(u + 1)
        cp[f"s{G-2}"].wait()
        cp[f"s{G-1}"].wait()

    plsc.subcore_barrier()
    pltpu.sync_copy(accum_sh.at[pl.ds(sid * RPT, RPT)],
                    out_hbm.at[cid, pl.ds(sid * RPT, RPT)])


BM = 2000


def _tc_a_body(dp_ref, x_ref, w_ref, g_ref, dis_ref):
    deg = dp_ref[0, :, 0] + dp_ref[1, :, 0] + 1.0
    dis = lax.rsqrt(deg)
    h = lax.dot_general(x_ref[...], w_ref[...], (((1,), (1,)), ((), ())),
                        preferred_element_type=jnp.float32)
    g_ref[...] = h * dis[:, None]
    dis_ref[...] = dis[:, None]


def _tc_a(degparts, x, w1):
    return pl.pallas_call(
        _tc_a_body,
        grid=(N // BM,),
        in_specs=[
            pl.BlockSpec((NC, BM, D), lambda i: (0, i, 0)),
            pl.BlockSpec((BM, D), lambda i: (i, 0)),
            pl.BlockSpec((D, D), lambda i: (0, 0)),
        ],
        out_specs=[
            pl.BlockSpec((BM, D), lambda i: (i, 0)),
            pl.BlockSpec((BM, 1), lambda i: (i, 0)),
        ],
        out_shape=[
            jax.ShapeDtypeStruct((N, D), jnp.float32),
            jax.ShapeDtypeStruct((N, 1), jnp.float32),
        ],
    )(degparts, x, w1)


def _tc_b_body(p_ref, g_ref, dis_ref, b_ref, w_ref, g2_ref):
    s = (p_ref[0] + p_ref[1] + g_ref[...]) * dis_ref[...] + b_ref[...]
    s = jnp.maximum(s, 0.0)
    h2 = lax.dot_general(s, w_ref[...], (((1,), (1,)), ((), ())),
                         preferred_element_type=jnp.float32)
    g2_ref[...] = h2 * dis_ref[...]


def _tc_b(parts, g, dis, b1, w2):
    return pl.pallas_call(
        _tc_b_body,
        grid=(N // BM,),
        in_specs=[
            pl.BlockSpec((NC, BM, D), lambda i: (0, i, 0)),
            pl.BlockSpec((BM, D), lambda i: (i, 0)),
            pl.BlockSpec((BM, 1), lambda i: (i, 0)),
            pl.BlockSpec((1, D), lambda i: (0, 0)),
            pl.BlockSpec((D, D), lambda i: (0, 0)),
        ],
        out_specs=pl.BlockSpec((BM, D), lambda i: (i, 0)),
        out_shape=jax.ShapeDtypeStruct((N, D), jnp.float32),
    )(parts, g, dis, b1, w2)


def _tc_c_body(p_ref, g_ref, dis_ref, b_ref, out_ref):
    out_ref[...] = (p_ref[0] + p_ref[1] + g_ref[...]) * dis_ref[...] + b_ref[...]


def _tc_c(parts, g, dis, b2):
    return pl.pallas_call(
        _tc_c_body,
        grid=(N // BM,),
        in_specs=[
            pl.BlockSpec((NC, BM, D), lambda i: (0, i, 0)),
            pl.BlockSpec((BM, D), lambda i: (i, 0)),
            pl.BlockSpec((BM, 1), lambda i: (i, 0)),
            pl.BlockSpec((1, D), lambda i: (0, 0)),
        ],
        out_specs=pl.BlockSpec((BM, D), lambda i: (i, 0)),
        out_shape=jax.ShapeDtypeStruct((N, D), jnp.float32),
    )(parts, g, dis, b2)


def kernel(x, edge_index, W1, b1, W2, b2):
    row = edge_index[0].astype(jnp.int32)
    col = edge_index[1].astype(jnp.int32)
    row_t = jnp.concatenate(
        [row, jnp.zeros((EPAD - E,), jnp.int32)]).reshape(NW, K, CHUNK)
    col_t = jnp.concatenate(
        [col, jnp.full((EPAD - E,), N, jnp.int32)]).reshape(NW, K, CHUNK)

    degparts = _deg_kernel(col_t)
    g1, dis = _tc_a(degparts, x, W1)
    parts1 = _agg_kernel(g1, row_t, col_t)
    g2 = _tc_b(parts1, g1, dis, b1.reshape(1, D), W2)
    parts2 = _agg_kernel(g2, row_t, col_t)
    return _tc_c(parts2, g2, dis, b2.reshape(1, D))


# speedup vs baseline: 1.1836x; 1.0441x over previous
import functools

import jax
import jax.numpy as jnp
from jax import lax
from jax.experimental import pallas as pl
from jax.experimental.pallas import tpu as pltpu
from jax.experimental.pallas import tpu_sc as plsc

N = 10000
D = 128
E = 320000
NC = 2
NS = 16
L = 16
CHUNK = 128
NW = NC * NS
K = 80
G = 16
EPAD = NW * K * CHUNK
NPAD = 10112
RPT = NPAD // NS

_mesh = plsc.VectorSubcoreMesh(core_axis_name="c", subcore_axis_name="s")


@functools.partial(
    pl.kernel,
    out_type=jax.ShapeDtypeStruct((NC, NPAD, D), jnp.float32),
    mesh=_mesh,
    scratch_types=[
        pltpu.VMEM((G, CHUNK), jnp.int32),
        pltpu.VMEM((CHUNK, D), jnp.float32),
        pltpu.VMEM_SHARED((NPAD, D), jnp.float32),
    ],
)
def _deg_kernel(col_hbm, out_hbm, coli_v, ones_v, accum_sh):
    cid = lax.axis_index("c")
    sid = lax.axis_index("s")
    wid = sid * NC + cid

    @pl.loop(0, CHUNK)
    def _(i):
        @pl.loop(0, D, step=L)
        def _(j):
            ones_v[i, pl.ds(j, L)] = jnp.zeros((L,), jnp.float32)

    @pl.loop(0, 4)
    def _(i):
        pltpu.sync_copy(ones_v,
                        accum_sh.at[pl.ds(sid * RPT + i * CHUNK, CHUNK)])
    pltpu.sync_copy(ones_v.at[pl.ds(0, RPT - 4 * CHUNK)],
                    accum_sh.at[pl.ds(sid * RPT + 4 * CHUNK, RPT - 4 * CHUNK)])
    plsc.subcore_barrier()

    @pl.loop(0, CHUNK)
    def _(i):
        @pl.loop(0, D, step=L)
        def _(j):
            ones_v[i, pl.ds(j, L)] = jnp.full((L,), 1.0, jnp.float32)

    @pl.loop(0, K, step=G)
    def _(j):
        jo = pl.multiple_of(j, G)
        pltpu.sync_copy(col_hbm.at[wid, pl.ds(jo, G)], coli_v)
        for u in range(G):
            pltpu.sync_copy(ones_v, accum_sh.at[coli_v.at[u]], add=True)

    plsc.subcore_barrier()
    pltpu.sync_copy(accum_sh.at[pl.ds(sid * RPT, RPT)],
                    out_hbm.at[cid, pl.ds(sid * RPT, RPT)])


@functools.partial(
    pl.kernel,
    out_type=jax.ShapeDtypeStruct((NC, NPAD, D), jnp.float32),
    mesh=_mesh,
    scratch_types=[
        pltpu.VMEM((G, CHUNK), jnp.int32),
        pltpu.VMEM((G, CHUNK), jnp.int32),
        pltpu.VMEM((2, CHUNK, D), jnp.float32),
        pltpu.VMEM_SHARED((NPAD, D), jnp.float32),
        pltpu.SemaphoreType.DMA((2,)),
        pltpu.SemaphoreType.DMA((2,)),
    ],
)
def _agg_kernel(g_hbm, row_hbm, col_hbm, out_hbm,
                rowi_v, coli_v, msg_v, accum_sh, gsem, ssem):
    cid = lax.axis_index("c")
    sid = lax.axis_index("s")
    wid = sid * NC + cid

    @pl.loop(0, CHUNK)
    def _(i):
        @pl.loop(0, D, step=L)
        def _(j):
            msg_v[0, i, pl.ds(j, L)] = jnp.zeros((L,), jnp.float32)

    @pl.loop(0, 4)
    def _(i):
        pltpu.sync_copy(msg_v.at[0],
                        accum_sh.at[pl.ds(sid * RPT + i * CHUNK, CHUNK)])
    pltpu.sync_copy(msg_v.at[0, pl.ds(0, RPT - 4 * CHUNK)],
                    accum_sh.at[pl.ds(sid * RPT + 4 * CHUNK, RPT - 4 * CHUNK)])
    plsc.subcore_barrier()

    @pl.loop(0, K, step=G)
    def _(j):
        jo = pl.multiple_of(j, G)
        pltpu.sync_copy(row_hbm.at[wid, pl.ds(jo, G)], rowi_v)
        pltpu.sync_copy(col_hbm.at[wid, pl.ds(jo, G)], coli_v)

        def gather(u):
            return pltpu.async_copy(g_hbm.at[rowi_v.at[u]],
                                    msg_v.at[u % 2], gsem.at[u % 2])

        def scatter(u):
            return pltpu.async_copy(msg_v.at[u % 2],
                                    accum_sh.at[coli_v.at[u]],
                                    ssem.at[u % 2], add=True)

        cp = {}
        cp["g0"] = gather(0)
        cp["g1"] = gather(1)
        cp["g0"].wait(); cp["s0"] = scatter(0)
        cp["g1"].wait(); cp["s1"] = scatter(1)
        for u in range(2, G, 2):
            cp[f"s{u-2}"].wait(); cp[f"g{u}"] = gather(u)
            cp[f"s{u-1}"].wait(); cp[f"g{u+1}"] = gather(u + 1)
            cp[f"g{u}"].wait(); cp[f"s{u}"] = scatter(u)
            cp[f"g{u+1}"].wait(); cp[f"s{u+1}"] = scatter(u + 1)
        cp[f"s{G-2}"].wait()
        cp[f"s{G-1}"].wait()

    plsc.subcore_barrier()
    pltpu.sync_copy(accum_sh.at[pl.ds(sid * RPT, RPT)],
                    out_hbm.at[cid, pl.ds(sid * RPT, RPT)])


BM = 2000


def _tc_a_body(dp_ref, x_ref, w_ref, g_ref, dis_ref):
    deg = dp_ref[0, :, 0] + dp_ref[1, :, 0] + 1.0
    dis = lax.rsqrt(deg)
    h = lax.dot_general(x_ref[...], w_ref[...], (((1,), (1,)), ((), ())),
                        preferred_element_type=jnp.float32)
    g_ref[...] = h * dis[:, None]
    dis_ref[...] = dis[:, None]


def _tc_a(degparts, x, w1):
    return pl.pallas_call(
        _tc_a_body,
        grid=(N // BM,),
        in_specs=[
            pl.BlockSpec((NC, BM, D), lambda i: (0, i, 0)),
            pl.BlockSpec((BM, D), lambda i: (i, 0)),
            pl.BlockSpec((D, D), lambda i: (0, 0)),
        ],
        out_specs=[
            pl.BlockSpec((BM, D), lambda i: (i, 0)),
            pl.BlockSpec((BM, 1), lambda i: (i, 0)),
        ],
        out_shape=[
            jax.ShapeDtypeStruct((N, D), jnp.float32),
            jax.ShapeDtypeStruct((N, 1), jnp.float32),
        ],
    )(degparts, x, w1)


def _tc_b_body(p_ref, g_ref, dis_ref, b_ref, w_ref, g2_ref):
    s = (p_ref[0] + p_ref[1] + g_ref[...]) * dis_ref[...] + b_ref[...]
    s = jnp.maximum(s, 0.0)
    h2 = lax.dot_general(s, w_ref[...], (((1,), (1,)), ((), ())),
                         preferred_element_type=jnp.float32)
    g2_ref[...] = h2 * dis_ref[...]


def _tc_b(parts, g, dis, b1, w2):
    return pl.pallas_call(
        _tc_b_body,
        grid=(N // BM,),
        in_specs=[
            pl.BlockSpec((NC, BM, D), lambda i: (0, i, 0)),
            pl.BlockSpec((BM, D), lambda i: (i, 0)),
            pl.BlockSpec((BM, 1), lambda i: (i, 0)),
            pl.BlockSpec((1, D), lambda i: (0, 0)),
            pl.BlockSpec((D, D), lambda i: (0, 0)),
        ],
        out_specs=pl.BlockSpec((BM, D), lambda i: (i, 0)),
        out_shape=jax.ShapeDtypeStruct((N, D), jnp.float32),
    )(parts, g, dis, b1, w2)


def _tc_c_body(p_ref, g_ref, dis_ref, b_ref, out_ref):
    out_ref[...] = (p_ref[0] + p_ref[1] + g_ref[...]) * dis_ref[...] + b_ref[...]


def _tc_c(parts, g, dis, b2):
    return pl.pallas_call(
        _tc_c_body,
        grid=(N // BM,),
        in_specs=[
            pl.BlockSpec((NC, BM, D), lambda i: (0, i, 0)),
            pl.BlockSpec((BM, D), lambda i: (i, 0)),
            pl.BlockSpec((BM, 1), lambda i: (i, 0)),
            pl.BlockSpec((1, D), lambda i: (0, 0)),
        ],
        out_specs=pl.BlockSpec((BM, D), lambda i: (i, 0)),
        out_shape=jax.ShapeDtypeStruct((N, D), jnp.float32),
    )(parts, g, dis, b2)


def kernel(x, edge_index, W1, b1, W2, b2):
    row = edge_index[0].astype(jnp.int32)
    col = edge_index[1].astype(jnp.int32)
    row_t = jnp.concatenate(
        [row, jnp.zeros((EPAD - E,), jnp.int32)]).reshape(NW, K, CHUNK)
    col_t = jnp.concatenate(
        [col, jnp.full((EPAD - E,), N, jnp.int32)]).reshape(NW, K, CHUNK)

    degparts = _deg_kernel(col_t)
    g1, dis = _tc_a(degparts, x, W1)
    parts1 = _agg_kernel(g1, row_t, col_t)
    g2 = _tc_b(parts1, g1, dis, b1.reshape(1, D), W2)
    parts2 = _agg_kernel(g2, row_t, col_t)
    return _tc_c(parts2, g2, dis, b2.reshape(1, D))


# speedup vs baseline: 1.2035x; 1.0168x over previous
import functools

import jax
import jax.numpy as jnp
from jax import lax
from jax.experimental import pallas as pl
from jax.experimental.pallas import tpu as pltpu
from jax.experimental.pallas import tpu_sc as plsc

N = 10000
D = 128
E = 320000
NC = 2
NS = 16
L = 16
CHUNK = 128
NW = NC * NS
K = 80
G = 40
EPAD = NW * K * CHUNK
NPAD = 10112
RPT = NPAD // NS

_mesh = plsc.VectorSubcoreMesh(core_axis_name="c", subcore_axis_name="s")


@functools.partial(
    pl.kernel,
    out_type=jax.ShapeDtypeStruct((NC, NPAD, D), jnp.float32),
    mesh=_mesh,
    scratch_types=[
        pltpu.VMEM((G, CHUNK), jnp.int32),
        pltpu.VMEM((CHUNK, D), jnp.float32),
        pltpu.VMEM_SHARED((NPAD, D), jnp.float32),
    ],
)
def _deg_kernel(col_hbm, out_hbm, coli_v, ones_v, accum_sh):
    cid = lax.axis_index("c")
    sid = lax.axis_index("s")
    wid = sid * NC + cid

    @pl.loop(0, CHUNK)
    def _(i):
        @pl.loop(0, D, step=L)
        def _(j):
            ones_v[i, pl.ds(j, L)] = jnp.zeros((L,), jnp.float32)

    @pl.loop(0, 4)
    def _(i):
        pltpu.sync_copy(ones_v,
                        accum_sh.at[pl.ds(sid * RPT + i * CHUNK, CHUNK)])
    pltpu.sync_copy(ones_v.at[pl.ds(0, RPT - 4 * CHUNK)],
                    accum_sh.at[pl.ds(sid * RPT + 4 * CHUNK, RPT - 4 * CHUNK)])
    plsc.subcore_barrier()

    @pl.loop(0, CHUNK)
    def _(i):
        @pl.loop(0, D, step=L)
        def _(j):
            ones_v[i, pl.ds(j, L)] = jnp.full((L,), 1.0, jnp.float32)

    @pl.loop(0, K, step=G)
    def _(j):
        jo = pl.multiple_of(j, G)
        pltpu.sync_copy(col_hbm.at[wid, pl.ds(jo, G)], coli_v)
        for u in range(G):
            pltpu.sync_copy(ones_v, accum_sh.at[coli_v.at[u]], add=True)

    plsc.subcore_barrier()
    pltpu.sync_copy(accum_sh.at[pl.ds(sid * RPT, RPT)],
                    out_hbm.at[cid, pl.ds(sid * RPT, RPT)])


@functools.partial(
    pl.kernel,
    out_type=jax.ShapeDtypeStruct((NC, NPAD, D), jnp.float32),
    mesh=_mesh,
    scratch_types=[
        pltpu.VMEM((G, CHUNK), jnp.int32),
        pltpu.VMEM((G, CHUNK), jnp.int32),
        pltpu.VMEM((2, CHUNK, D), jnp.float32),
        pltpu.VMEM_SHARED((NPAD, D), jnp.float32),
        pltpu.SemaphoreType.DMA((2,)),
        pltpu.SemaphoreType.DMA((2,)),
    ],
)
def _agg_kernel(g_hbm, row_hbm, col_hbm, out_hbm,
                rowi_v, coli_v, msg_v, accum_sh, gsem, ssem):
    cid = lax.axis_index("c")
    sid = lax.axis_index("s")
    wid = sid * NC + cid

    @pl.loop(0, CHUNK)
    def _(i):
        @pl.loop(0, D, step=L)
        def _(j):
            msg_v[0, i, pl.ds(j, L)] = jnp.zeros((L,), jnp.float32)

    @pl.loop(0, 4)
    def _(i):
        pltpu.sync_copy(msg_v.at[0],
                        accum_sh.at[pl.ds(sid * RPT + i * CHUNK, CHUNK)])
    pltpu.sync_copy(msg_v.at[0, pl.ds(0, RPT - 4 * CHUNK)],
                    accum_sh.at[pl.ds(sid * RPT + 4 * CHUNK, RPT - 4 * CHUNK)])
    plsc.subcore_barrier()

    @pl.loop(0, K, step=G)
    def _(j):
        jo = pl.multiple_of(j, G)
        pltpu.sync_copy(row_hbm.at[wid, pl.ds(jo, G)], rowi_v)
        pltpu.sync_copy(col_hbm.at[wid, pl.ds(jo, G)], coli_v)

        def gather(u):
            return pltpu.async_copy(g_hbm.at[rowi_v.at[u]],
                                    msg_v.at[u % 2], gsem.at[u % 2])

        def scatter(u):
            return pltpu.async_copy(msg_v.at[u % 2],
                                    accum_sh.at[coli_v.at[u]],
                                    ssem.at[u % 2], add=True)

        cp = {}
        cp["g0"] = gather(0)
        cp["g1"] = gather(1)
        cp["g0"].wait(); cp["s0"] = scatter(0)
        cp["g1"].wait(); cp["s1"] = scatter(1)
        for u in range(2, G, 2):
            cp[f"s{u-2}"].wait(); cp[f"g{u}"] = gather(u)
            cp[f"s{u-1}"].wait(); cp[f"g{u+1}"] = gather(u + 1)
            cp[f"g{u}"].wait(); cp[f"s{u}"] = scatter(u)
            cp[f"g{u+1}"].wait(); cp[f"s{u+1}"] = scatter(u + 1)
        cp[f"s{G-2}"].wait()
        cp[f"s{G-1}"].wait()

    plsc.subcore_barrier()
    pltpu.sync_copy(accum_sh.at[pl.ds(sid * RPT, RPT)],
                    out_hbm.at[cid, pl.ds(sid * RPT, RPT)])


BM = 2000


def _tc_a_body(dp_ref, x_ref, w_ref, g_ref, dis_ref):
    deg = dp_ref[0, :, 0] + dp_ref[1, :, 0] + 1.0
    dis = lax.rsqrt(deg)
    h = lax.dot_general(x_ref[...], w_ref[...], (((1,), (1,)), ((), ())),
                        preferred_element_type=jnp.float32)
    g_ref[...] = h * dis[:, None]
    dis_ref[...] = dis[:, None]


def _tc_a(degparts, x, w1):
    return pl.pallas_call(
        _tc_a_body,
        grid=(N // BM,),
        in_specs=[
            pl.BlockSpec((NC, BM, D), lambda i: (0, i, 0)),
            pl.BlockSpec((BM, D), lambda i: (i, 0)),
            pl.BlockSpec((D, D), lambda i: (0, 0)),
        ],
        out_specs=[
            pl.BlockSpec((BM, D), lambda i: (i, 0)),
            pl.BlockSpec((BM, 1), lambda i: (i, 0)),
        ],
        out_shape=[
            jax.ShapeDtypeStruct((N, D), jnp.float32),
            jax.ShapeDtypeStruct((N, 1), jnp.float32),
        ],
    )(degparts, x, w1)


def _tc_b_body(p_ref, g_ref, dis_ref, b_ref, w_ref, g2_ref):
    s = (p_ref[0] + p_ref[1] + g_ref[...]) * dis_ref[...] + b_ref[...]
    s = jnp.maximum(s, 0.0)
    h2 = lax.dot_general(s, w_ref[...], (((1,), (1,)), ((), ())),
                         preferred_element_type=jnp.float32)
    g2_ref[...] = h2 * dis_ref[...]


def _tc_b(parts, g, dis, b1, w2):
    return pl.pallas_call(
        _tc_b_body,
        grid=(N // BM,),
        in_specs=[
            pl.BlockSpec((NC, BM, D), lambda i: (0, i, 0)),
            pl.BlockSpec((BM, D), lambda i: (i, 0)),
            pl.BlockSpec((BM, 1), lambda i: (i, 0)),
            pl.BlockSpec((1, D), lambda i: (0, 0)),
            pl.BlockSpec((D, D), lambda i: (0, 0)),
        ],
        out_specs=pl.BlockSpec((BM, D), lambda i: (i, 0)),
        out_shape=jax.ShapeDtypeStruct((N, D), jnp.float32),
    )(parts, g, dis, b1, w2)


def _tc_c_body(p_ref, g_ref, dis_ref, b_ref, out_ref):
    out_ref[...] = (p_ref[0] + p_ref[1] + g_ref[...]) * dis_ref[...] + b_ref[...]


def _tc_c(parts, g, dis, b2):
    return pl.pallas_call(
        _tc_c_body,
        grid=(N // BM,),
        in_specs=[
            pl.BlockSpec((NC, BM, D), lambda i: (0, i, 0)),
            pl.BlockSpec((BM, D), lambda i: (i, 0)),
            pl.BlockSpec((BM, 1), lambda i: (i, 0)),
            pl.BlockSpec((1, D), lambda i: (0, 0)),
        ],
        out_specs=pl.BlockSpec((BM, D), lambda i: (i, 0)),
        out_shape=jax.ShapeDtypeStruct((N, D), jnp.float32),
    )(parts, g, dis, b2)


def kernel(x, edge_index, W1, b1, W2, b2):
    row = edge_index[0].astype(jnp.int32)
    col = edge_index[1].astype(jnp.int32)
    row_t = jnp.concatenate(
        [row, jnp.zeros((EPAD - E,), jnp.int32)]).reshape(NW, K, CHUNK)
    col_t = jnp.concatenate(
        [col, jnp.full((EPAD - E,), N, jnp.int32)]).reshape(NW, K, CHUNK)

    degparts = _deg_kernel(col_t)
    g1, dis = _tc_a(degparts, x, W1)
    parts1 = _agg_kernel(g1, row_t, col_t)
    g2 = _tc_b(parts1, g1, dis, b1.reshape(1, D), W2)
    parts2 = _agg_kernel(g2, row_t, col_t)
    return _tc_c(parts2, g2, dis, b2.reshape(1, D))
